# baseline (speedup 1.0000x reference)
import functools

import jax
import jax.numpy as jnp
from jax import lax
from jax.experimental import pallas as pl
from jax.experimental.pallas import tpu as pltpu
from jax.experimental.pallas import tpu_sc as plsc

L = 16


def _sc_conv(x_flat, cu_pad, w, n_starts):
    T, D = x_flat.shape
    W = w.shape[0]
    info = plsc.get_sparse_core_info()
    NC, NS = info.num_cores, info.num_subcores
    NW = NC * NS
    TPW = T // NW
    TILE = 32
    NT = TPW // TILE
    CG = D // L
    CU = cu_pad.shape[0]
    H = 8

    mesh = plsc.VectorSubcoreMesh(core_axis_name="c", subcore_axis_name="s")

    @functools.partial(
        pl.kernel,
        mesh=mesh,
        out_type=jax.ShapeDtypeStruct((T, D), jnp.float32),
        scratch_types=[
            pltpu.VMEM((TILE + H, D), jnp.float32),
            pltpu.VMEM((TILE, D), jnp.float32),
            pltpu.VMEM((W, D), jnp.float32),
            pltpu.VMEM((CU,), jnp.int32),
        ],
    )
    def k(x_hbm, cu_hbm, w_hbm, out_hbm, xbuf, ybuf, wv, cus):
        wid = lax.axis_index("s") * NC + lax.axis_index("c")
        base = wid * TPW

        pltpu.sync_copy(w_hbm, wv)
        pltpu.sync_copy(cu_hbm, cus)

        def sread(ref, i):
            return ref[pl.ds(i, L)][0]

        zero = jnp.zeros((L,), jnp.float32)

        def zbody(i, _):
            xbuf[i // CG, pl.ds((i % CG) * L, L)] = zero
            return 0

        lax.fori_loop(0, H * CG, zbody, 0)

        def tile_body(it, _):
            start = base + it * TILE

            @pl.when(start >= H)
            def _():
                pltpu.sync_copy(x_hbm.at[pl.ds(start - H, H)],
                                xbuf.at[pl.ds(0, H)])

            pltpu.sync_copy(x_hbm.at[pl.ds(start, TILE)],
                            xbuf.at[pl.ds(H, TILE)])

            @plsc.parallel_loop(0, CG, 1, unroll=2)
            def cg_body(cg):
                col = cg * L
                w0 = wv[0, pl.ds(col, L)]
                w1 = wv[1, pl.ds(col, L)]
                w2 = wv[2, pl.ds(col, L)]
                w3 = wv[3, pl.ds(col, L)]
                xs = [xbuf[r + H - 3, pl.ds(col, L)]
                      for r in range(TILE + 3)]
                for t in range(TILE):
                    acc = ((xs[t] * w0 + xs[t + 1] * w1)
                           + (xs[t + 2] * w2 + xs[t + 3] * w3))
                    ybuf[t, pl.ds(col, L)] = acc / (1.0 + jnp.exp(-acc))

            def fix_body(bi, _):
                cval = sread(cus, bi)
                for dt in range(W - 1):
                    t = cval + dt
                    pred = (t >= start) & (t < start + TILE)

                    @pl.when(pred)
                    def _():
                        def dcount(i, dc):
                            ci = sread(cus, i)
                            ok = (ci <= t) & (ci != sread(cus, i - 1))
                            return dc + jnp.where(ok, 1, 0)

                        d = lax.fori_loop(1, n_starts, dcount, jnp.int32(1))
                        s = sread(cus, d - 1)
                        m = [jnp.where(t - (W - 1) + j >= s, 1.0, 0.0).astype(
                            jnp.float32) for j in range(W)]
                        row = t - start

                        def cg_fix(cg, _):
                            col = cg * L
                            r0 = row + H - 3
                            acc = ((xbuf[r0 + 0, pl.ds(col, L)]
                                    * wv[0, pl.ds(col, L)]) * m[0]
                                   + (xbuf[r0 + 1, pl.ds(col, L)]
                                      * wv[1, pl.ds(col, L)]) * m[1]
                                   + (xbuf[r0 + 2, pl.ds(col, L)]
                                      * wv[2, pl.ds(col, L)]) * m[2]
                                   + (xbuf[r0 + 3, pl.ds(col, L)]
                                      * wv[3, pl.ds(col, L)]) * m[3])
                            ybuf[row, pl.ds(col, L)] = (
                                acc / (1.0 + jnp.exp(-acc)))
                            return 0

                        lax.fori_loop(0, CG, cg_fix, 0)
                return 0

            lax.fori_loop(1, n_starts, fix_body, 0)

            pltpu.sync_copy(ybuf, out_hbm.at[pl.ds(start, TILE)])
            return 0

        lax.fori_loop(0, NT, tile_body, 0)

    return k(x_flat, cu_pad, w)


def kernel(x, cu_seqlens, kernel):
    B, T, D = x.shape
    W = kernel.shape[0]
    x_flat = x[0]
    w = kernel.reshape(W, D).astype(jnp.float32)
    n_starts = cu_seqlens.shape[0] - 1
    CU = 32
    cu_pad = jnp.concatenate(
        [cu_seqlens.astype(jnp.int32),
         jnp.full((CU - cu_seqlens.shape[0],), T, dtype=jnp.int32)])
    y = _sc_conv(x_flat.astype(jnp.float32), cu_pad, w, n_starts)
    return y.astype(x.dtype)[None]


# speedup vs baseline: 1.1767x; 1.1767x over previous
import functools

import jax
import jax.numpy as jnp
from jax import lax
from jax.experimental import pallas as pl
from jax.experimental.pallas import tpu as pltpu
from jax.experimental.pallas import tpu_sc as plsc

L = 16


def _sc_conv(x_flat, cu_pad, w, n_starts):
    T, D = x_flat.shape
    W = w.shape[0]
    info = plsc.get_sparse_core_info()
    NC, NS = info.num_cores, info.num_subcores
    NW = NC * NS
    TPW = T // NW
    TILE = 32
    NT = TPW // TILE
    CG = D // L
    CU = cu_pad.shape[0]
    H = 8

    mesh = plsc.VectorSubcoreMesh(core_axis_name="c", subcore_axis_name="s")

    @functools.partial(
        pl.kernel,
        mesh=mesh,
        out_type=jax.ShapeDtypeStruct((T, D), jnp.float32),
        scratch_types=[
            pltpu.VMEM((TILE + H, D), jnp.float32),
            pltpu.VMEM((TILE, D), jnp.float32),
            pltpu.VMEM((W, D), jnp.float32),
            pltpu.VMEM((CU,), jnp.int32),
        ],
    )
    def k(x_hbm, cu_hbm, w_hbm, out_hbm, xbuf, ybuf, wv, cus):
        wid = lax.axis_index("s") * NC + lax.axis_index("c")
        base = wid * TPW

        pltpu.sync_copy(w_hbm, wv)
        pltpu.sync_copy(cu_hbm, cus)

        def sread(ref, i):
            return ref[pl.ds(i, L)][0]

        zero = jnp.zeros((L,), jnp.float32)

        def zbody(i, _):
            xbuf[i // CG, pl.ds((i % CG) * L, L)] = zero
            return 0

        lax.fori_loop(0, H * CG, zbody, 0)

        def tile_body(it, _):
            start = base + it * TILE

            @pl.when(start >= H)
            def _():
                pltpu.sync_copy(x_hbm.at[pl.ds(start - H, H)],
                                xbuf.at[pl.ds(0, H)])

            pltpu.sync_copy(x_hbm.at[pl.ds(start, TILE)],
                            xbuf.at[pl.ds(H, TILE)])

            @plsc.parallel_loop(0, CG, 1)
            def cg_body(cg):
                col = cg * L
                w0 = wv[0, pl.ds(col, L)]
                w1 = wv[1, pl.ds(col, L)]
                w2 = wv[2, pl.ds(col, L)]
                w3 = wv[3, pl.ds(col, L)]
                xs = [xbuf[r + H - 3, pl.ds(col, L)]
                      for r in range(TILE + 3)]
                for t in range(TILE):
                    acc = ((xs[t] * w0 + xs[t + 1] * w1)
                           + (xs[t + 2] * w2 + xs[t + 3] * w3))
                    ybuf[t, pl.ds(col, L)] = acc / (1.0 + jnp.exp(-acc))

            def fix_body(bi, _):
                cval = sread(cus, bi)
                for dt in range(W - 1):
                    t = cval + dt
                    pred = (t >= start) & (t < start + TILE)

                    @pl.when(pred)
                    def _():
                        def dcount(i, dc):
                            ci = sread(cus, i)
                            ok = (ci <= t) & (ci != sread(cus, i - 1))
                            return dc + jnp.where(ok, 1, 0)

                        d = lax.fori_loop(1, n_starts, dcount, jnp.int32(1))
                        s = sread(cus, d - 1)
                        m = [jnp.where(t - (W - 1) + j >= s, 1.0, 0.0).astype(
                            jnp.float32) for j in range(W)]
                        row = t - start

                        def cg_fix(cg, _):
                            col = cg * L
                            r0 = row + H - 3
                            acc = ((xbuf[r0 + 0, pl.ds(col, L)]
                                    * wv[0, pl.ds(col, L)]) * m[0]
                                   + (xbuf[r0 + 1, pl.ds(col, L)]
                                      * wv[1, pl.ds(col, L)]) * m[1]
                                   + (xbuf[r0 + 2, pl.ds(col, L)]
                                      * wv[2, pl.ds(col, L)]) * m[2]
                                   + (xbuf[r0 + 3, pl.ds(col, L)]
                                      * wv[3, pl.ds(col, L)]) * m[3])
                            ybuf[row, pl.ds(col, L)] = (
                                acc / (1.0 + jnp.exp(-acc)))
                            return 0

                        lax.fori_loop(0, CG, cg_fix, 0)
                return 0

            lax.fori_loop(1, n_starts, fix_body, 0)

            pltpu.sync_copy(ybuf, out_hbm.at[pl.ds(start, TILE)])
            return 0

        lax.fori_loop(0, NT, tile_body, 0)

    return k(x_flat, cu_pad, w)


def kernel(x, cu_seqlens, kernel):
    B, T, D = x.shape
    W = kernel.shape[0]
    x_flat = x[0]
    w = kernel.reshape(W, D).astype(jnp.float32)
    n_starts = cu_seqlens.shape[0] - 1
    CU = 32
    cu_pad = jnp.concatenate(
        [cu_seqlens.astype(jnp.int32),
         jnp.full((CU - cu_seqlens.shape[0],), T, dtype=jnp.int32)])
    y = _sc_conv(x_flat.astype(jnp.float32), cu_pad, w, n_starts)
    return y.astype(x.dtype)[None]


# speedup vs baseline: 1.6043x; 1.3634x over previous
import functools

import jax
import jax.numpy as jnp
from jax import lax
from jax.experimental import pallas as pl
from jax.experimental.pallas import tpu as pltpu
from jax.experimental.pallas import tpu_sc as plsc

L = 16


def _sc_conv(x_flat, cu_pad, w, n_starts):
    T, D = x_flat.shape
    W = w.shape[0]
    info = plsc.get_sparse_core_info()
    NC, NS = info.num_cores, info.num_subcores
    NW = NC * NS
    TPW = T // NW
    TILE = 32
    NT = TPW // TILE
    CG = D // L
    CU = cu_pad.shape[0]
    H = 8

    mesh = plsc.VectorSubcoreMesh(core_axis_name="c", subcore_axis_name="s")

    @functools.partial(
        pl.kernel,
        mesh=mesh,
        out_type=jax.ShapeDtypeStruct((T, D), jnp.float32),
        scratch_types=[
            pltpu.VMEM((TILE + H, D), jnp.float32),
            pltpu.VMEM((TILE, D), jnp.float32),
            pltpu.VMEM((W, D), jnp.float32),
            pltpu.VMEM((CU,), jnp.int32),
        ],
    )
    def k(x_hbm, cu_hbm, w_hbm, out_hbm, xbuf, ybuf, wv, cus):
        wid = lax.axis_index("s") * NC + lax.axis_index("c")
        base = wid * TPW

        pltpu.sync_copy(w_hbm, wv)
        pltpu.sync_copy(cu_hbm, cus)

        def sread(ref, i):
            return ref[pl.ds(i, L)][0]

        zero = jnp.zeros((L,), jnp.float32)

        def zbody(i, _):
            xbuf[i // CG, pl.ds((i % CG) * L, L)] = zero
            return 0

        lax.fori_loop(0, H * CG, zbody, 0)

        def tile_body(it, _):
            start = base + it * TILE

            @pl.when(start >= H)
            def _():
                pltpu.sync_copy(x_hbm.at[pl.ds(start - H, H)],
                                xbuf.at[pl.ds(0, H)])

            pltpu.sync_copy(x_hbm.at[pl.ds(start, TILE)],
                            xbuf.at[pl.ds(H, TILE)])

            @plsc.parallel_loop(0, CG, 1)
            def cg_body(cg):
                col = cg * L
                w0 = wv[0, pl.ds(col, L)]
                w1 = wv[1, pl.ds(col, L)]
                w2 = wv[2, pl.ds(col, L)]
                w3 = wv[3, pl.ds(col, L)]
                x0 = xbuf[H - 3, pl.ds(col, L)]
                x1 = xbuf[H - 2, pl.ds(col, L)]
                x2 = xbuf[H - 1, pl.ds(col, L)]
                for t in range(TILE):
                    x3 = xbuf[t + H, pl.ds(col, L)]
                    acc = (x0 * w0 + x1 * w1) + (x2 * w2 + x3 * w3)
                    ybuf[t, pl.ds(col, L)] = acc / (1.0 + jnp.exp(-acc))
                    x0, x1, x2 = x1, x2, x3

            def fix_body(bi, _):
                cval = sread(cus, bi)
                for dt in range(W - 1):
                    t = cval + dt
                    pred = (t >= start) & (t < start + TILE)

                    @pl.when(pred)
                    def _():
                        def dcount(i, dc):
                            ci = sread(cus, i)
                            ok = (ci <= t) & (ci != sread(cus, i - 1))
                            return dc + jnp.where(ok, 1, 0)

                        d = lax.fori_loop(1, n_starts, dcount, jnp.int32(1))
                        s = sread(cus, d - 1)
                        m = [jnp.where(t - (W - 1) + j >= s, 1.0, 0.0).astype(
                            jnp.float32) for j in range(W)]
                        row = t - start

                        def cg_fix(cg, _):
                            col = cg * L
                            r0 = row + H - 3
                            acc = ((xbuf[r0 + 0, pl.ds(col, L)]
                                    * wv[0, pl.ds(col, L)]) * m[0]
                                   + (xbuf[r0 + 1, pl.ds(col, L)]
                                      * wv[1, pl.ds(col, L)]) * m[1]
                                   + (xbuf[r0 + 2, pl.ds(col, L)]
                                      * wv[2, pl.ds(col, L)]) * m[2]
                                   + (xbuf[r0 + 3, pl.ds(col, L)]
                                      * wv[3, pl.ds(col, L)]) * m[3])
                            ybuf[row, pl.ds(col, L)] = (
                                acc / (1.0 + jnp.exp(-acc)))
                            return 0

                        lax.fori_loop(0, CG, cg_fix, 0)
                return 0

            lax.fori_loop(1, n_starts, fix_body, 0)

            pltpu.sync_copy(ybuf, out_hbm.at[pl.ds(start, TILE)])
            return 0

        lax.fori_loop(0, NT, tile_body, 0)

    return k(x_flat, cu_pad, w)


def kernel(x, cu_seqlens, kernel):
    B, T, D = x.shape
    W = kernel.shape[0]
    x_flat = x[0]
    w = kernel.reshape(W, D).astype(jnp.float32)
    n_starts = cu_seqlens.shape[0] - 1
    CU = 32
    cu_pad = jnp.concatenate(
        [cu_seqlens.astype(jnp.int32),
         jnp.full((CU - cu_seqlens.shape[0],), T, dtype=jnp.int32)])
    y = _sc_conv(x_flat.astype(jnp.float32), cu_pad, w, n_starts)
    return y.astype(x.dtype)[None]
